# Initial kernel scaffold; baseline (speedup 1.0000x reference)
import jax
import jax.numpy as jnp
from jax.experimental import pallas as pl


def kernel(hidden_states, W):
    raise NotImplementedError("write your pallas kernel here")


# speedup vs baseline: 1.2694x; 1.2694x over previous
import jax
import jax.numpy as jnp
from jax.experimental import pallas as pl
from jax.experimental.pallas import tpu as pltpu

_E = 64
_K = 8
_BLOCK = 512


def _router_block(x_ref, w_ref, logits_ref, weights_ref, experts_ref):
    x = x_ref[...].astype(jnp.bfloat16)
    w = w_ref[...].astype(jnp.bfloat16)
    logits = jax.lax.dot_general(
        x, w, (((1,), (0,)), ((), ())), preferred_element_type=jnp.float32
    )
    logits_ref[...] = logits

    t = logits.shape[0]
    iota = jax.lax.broadcasted_iota(jnp.int32, (t, _E), 1)
    work = logits
    vals, idxs = [], []
    for _ in range(_K):
        m = jnp.max(work, axis=1, keepdims=True)
        idx = jnp.min(jnp.where(work == m, iota, _E), axis=1, keepdims=True)
        vals.append(m)
        idxs.append(idx)
        work = jnp.where(iota == idx, -jnp.inf, work)
    g = jnp.concatenate(vals, axis=1)
    e = jnp.concatenate(idxs, axis=1)
    ew = jnp.exp(g - g[:, :1])
    weights_ref[...] = ew / jnp.sum(ew, axis=1, keepdims=True)
    experts_ref[...] = e


def kernel(hidden_states, W):
    b, s, d = hidden_states.shape
    n = b * s
    x = hidden_states.reshape(n, d)
    grid = (n // _BLOCK,)
    logits, weights, experts = pl.pallas_call(
        _router_block,
        grid=grid,
        in_specs=[
            pl.BlockSpec((_BLOCK, d), lambda i: (i, 0)),
            pl.BlockSpec((d, _E), lambda i: (0, 0)),
        ],
        out_specs=[
            pl.BlockSpec((_BLOCK, _E), lambda i: (i, 0)),
            pl.BlockSpec((_BLOCK, _K), lambda i: (i, 0)),
            pl.BlockSpec((_BLOCK, _K), lambda i: (i, 0)),
        ],
        out_shape=[
            jax.ShapeDtypeStruct((n, _E), jnp.float32),
            jax.ShapeDtypeStruct((n, _K), jnp.float32),
            jax.ShapeDtypeStruct((n, _K), jnp.int32),
        ],
        compiler_params=pltpu.CompilerParams(
            dimension_semantics=("arbitrary",),
        ),
    )(x, W)
    return (weights, experts, logits)


# speedup vs baseline: 1.4582x; 1.1487x over previous
import jax
import jax.numpy as jnp
from jax.experimental import pallas as pl
from jax.experimental.pallas import tpu as pltpu

_E = 64
_K = 8
_BLOCK = 512


def _router_block(x_ref, w_ref, logits_ref, weights_ref, experts_ref):
    x = x_ref[...].astype(jnp.bfloat16)
    w = w_ref[...].astype(jnp.bfloat16)
    logits = jax.lax.dot_general(
        x, w, (((1,), (0,)), ((), ())), preferred_element_type=jnp.float32
    )
    logits_ref[...] = logits

    t = logits.shape[0]
    _C = 64
    iota_f = jax.lax.broadcasted_iota(jnp.int32, (_C, _E), 1).astype(jnp.float32)
    g_chunks, e_chunks = [], []
    for c in range(t // _C):
        work = logits[c * _C:(c + 1) * _C, :]
        vals, idxs = [], []
        for _ in range(_K):
            m = jnp.max(work, axis=1, keepdims=True)
            idx = jnp.min(jnp.where(work == m, iota_f, float(_E)),
                          axis=1, keepdims=True)
            vals.append(m)
            idxs.append(idx)
            work = jnp.where(iota_f == idx, -jnp.inf, work)
        g_chunks.append(jnp.concatenate(vals, axis=1))
        e_chunks.append(jnp.concatenate(idxs, axis=1))
    g = jnp.concatenate(g_chunks, axis=0)
    e = jnp.concatenate(e_chunks, axis=0)
    ew = jnp.exp(g - g[:, :1])
    weights_ref[...] = ew / jnp.sum(ew, axis=1, keepdims=True)
    experts_ref[...] = e.astype(jnp.int32)


def kernel(hidden_states, W):
    b, s, d = hidden_states.shape
    n = b * s
    x = hidden_states.reshape(n, d)
    grid = (n // _BLOCK,)
    logits, weights, experts = pl.pallas_call(
        _router_block,
        grid=grid,
        in_specs=[
            pl.BlockSpec((_BLOCK, d), lambda i: (i, 0)),
            pl.BlockSpec((d, _E), lambda i: (0, 0)),
        ],
        out_specs=[
            pl.BlockSpec((_BLOCK, _E), lambda i: (i, 0)),
            pl.BlockSpec((_BLOCK, _K), lambda i: (i, 0)),
            pl.BlockSpec((_BLOCK, _K), lambda i: (i, 0)),
        ],
        out_shape=[
            jax.ShapeDtypeStruct((n, _E), jnp.float32),
            jax.ShapeDtypeStruct((n, _K), jnp.float32),
            jax.ShapeDtypeStruct((n, _K), jnp.int32),
        ],
        compiler_params=pltpu.CompilerParams(
            dimension_semantics=("arbitrary",),
        ),
    )(x, W)
    return (weights, experts, logits)


# speedup vs baseline: 1.7557x; 1.2040x over previous
import jax
import jax.numpy as jnp
from jax.experimental import pallas as pl
from jax.experimental.pallas import tpu as pltpu

_E = 64
_K = 8
_BLOCK = 512


def _router_block(x_ref, w_ref, logits_ref, weights_ref, experts_ref):
    x = x_ref[...].astype(jnp.bfloat16)
    w = w_ref[...].astype(jnp.bfloat16)
    logits = jax.lax.dot_general(
        x, w, (((1,), (0,)), ((), ())), preferred_element_type=jnp.float32
    )
    logits_ref[...] = logits

    t = logits.shape[0]
    lt = logits.T
    _C = 128
    iota_f = jax.lax.broadcasted_iota(jnp.int32, (_E, _C), 0).astype(jnp.float32)
    wt_cols, et_cols = [], []
    for c in range(t // _C):
        work = lt[:, c * _C:(c + 1) * _C]
        vals, idxs = [], []
        for _ in range(_K):
            m = jnp.max(work, axis=0, keepdims=True)
            idx = jnp.min(jnp.where(work == m, iota_f, float(_E)),
                          axis=0, keepdims=True)
            vals.append(m)
            idxs.append(idx)
            work = jnp.where(iota_f == idx, -jnp.inf, work)
        g = jnp.concatenate(vals, axis=0)
        e = jnp.concatenate(idxs, axis=0)
        ew = jnp.exp(g - g[0:1, :])
        wt_cols.append(ew / jnp.sum(ew, axis=0, keepdims=True))
        et_cols.append(e)
    w_t = jnp.concatenate(wt_cols, axis=1)
    e_t = jnp.concatenate(et_cols, axis=1)
    weights_ref[...] = w_t.T
    experts_ref[...] = e_t.T.astype(jnp.int32)


def kernel(hidden_states, W):
    b, s, d = hidden_states.shape
    n = b * s
    x = hidden_states.reshape(n, d)
    grid = (n // _BLOCK,)
    logits, weights, experts = pl.pallas_call(
        _router_block,
        grid=grid,
        in_specs=[
            pl.BlockSpec((_BLOCK, d), lambda i: (i, 0)),
            pl.BlockSpec((d, _E), lambda i: (0, 0)),
        ],
        out_specs=[
            pl.BlockSpec((_BLOCK, _E), lambda i: (i, 0)),
            pl.BlockSpec((_BLOCK, _K), lambda i: (i, 0)),
            pl.BlockSpec((_BLOCK, _K), lambda i: (i, 0)),
        ],
        out_shape=[
            jax.ShapeDtypeStruct((n, _E), jnp.float32),
            jax.ShapeDtypeStruct((n, _K), jnp.float32),
            jax.ShapeDtypeStruct((n, _K), jnp.int32),
        ],
        compiler_params=pltpu.CompilerParams(
            dimension_semantics=("arbitrary",),
        ),
    )(x, W)
    return (weights, experts, logits)


# speedup vs baseline: 1.8144x; 1.0334x over previous
import jax
import jax.numpy as jnp
from jax.experimental import pallas as pl
from jax.experimental.pallas import tpu as pltpu

_E = 64
_K = 8
_BLOCK = 1024


def _router_block(x_ref, w_ref, logits_ref, weights_ref, experts_ref):
    x = x_ref[...].astype(jnp.bfloat16)
    w = w_ref[...].astype(jnp.bfloat16)
    logits = jax.lax.dot_general(
        x, w, (((1,), (0,)), ((), ())), preferred_element_type=jnp.float32
    )
    logits_ref[...] = logits

    t = logits.shape[0]
    lt = logits.T
    _C = 128
    iota_f = jax.lax.broadcasted_iota(jnp.int32, (_E, _C), 0).astype(jnp.float32)
    wt_cols, et_cols = [], []
    for c in range(t // _C):
        work = lt[:, c * _C:(c + 1) * _C]
        vals, idxs = [], []
        for _ in range(_K):
            m = jnp.max(work, axis=0, keepdims=True)
            idx = jnp.min(jnp.where(work == m, iota_f, float(_E)),
                          axis=0, keepdims=True)
            vals.append(m)
            idxs.append(idx)
            work = jnp.where(iota_f == idx, -jnp.inf, work)
        g = jnp.concatenate(vals, axis=0)
        e = jnp.concatenate(idxs, axis=0)
        ew = jnp.exp(g - g[0:1, :])
        wt_cols.append(ew / jnp.sum(ew, axis=0, keepdims=True))
        et_cols.append(e)
    w_t = jnp.concatenate(wt_cols, axis=1)
    e_t = jnp.concatenate(et_cols, axis=1)
    weights_ref[...] = w_t.T
    experts_ref[...] = e_t.T.astype(jnp.int32)


def kernel(hidden_states, W):
    b, s, d = hidden_states.shape
    n = b * s
    x = hidden_states.reshape(n, d)
    grid = (n // _BLOCK,)
    logits, weights, experts = pl.pallas_call(
        _router_block,
        grid=grid,
        in_specs=[
            pl.BlockSpec((_BLOCK, d), lambda i: (i, 0)),
            pl.BlockSpec((d, _E), lambda i: (0, 0)),
        ],
        out_specs=[
            pl.BlockSpec((_BLOCK, _E), lambda i: (i, 0)),
            pl.BlockSpec((_BLOCK, _K), lambda i: (i, 0)),
            pl.BlockSpec((_BLOCK, _K), lambda i: (i, 0)),
        ],
        out_shape=[
            jax.ShapeDtypeStruct((n, _E), jnp.float32),
            jax.ShapeDtypeStruct((n, _K), jnp.float32),
            jax.ShapeDtypeStruct((n, _K), jnp.int32),
        ],
        compiler_params=pltpu.CompilerParams(
            dimension_semantics=("arbitrary",),
        ),
    )(x, W)
    return (weights, experts, logits)
